# baseline (speedup 1.0000x reference)
import functools

import jax
import jax.numpy as jnp
from jax import lax
from jax.experimental import pallas as pl
from jax.experimental.pallas import tpu as pltpu
from jax.experimental.pallas import tpu_sc as plsc

_L = 16


@functools.lru_cache(maxsize=None)
def _build(B, N_FEA, V, D):
    info = plsc.get_sparse_core_info()
    NC, NS = info.num_cores, info.num_subcores
    NW = NC * NS
    assert B % (8 * NW) == 0
    b_per_w = B // NW
    G = b_per_w // _L
    mesh = plsc.VectorSubcoreMesh(core_axis_name="c", subcore_axis_name="s")

    @functools.partial(
        pl.kernel,
        mesh=mesh,
        out_type=jax.ShapeDtypeStruct((D, B), jnp.float32),
        scratch_types=[
            pltpu.VMEM((8, b_per_w), jnp.int32),
            pltpu.SMEM((1, b_per_w), jnp.int32),
            pltpu.VMEM((_L, 4, D // 4, 128), jnp.float32),
            pltpu.VMEM((D, b_per_w), jnp.float32),
            pltpu.SemaphoreType.DMA((_L,)),
        ],
    )
    def k(ufT_hbm, tT_hbm, outT_hbm, uf_v, idx_s, blk_v, w_v, sems):
        wid = lax.axis_index("s") * NC + lax.axis_index("c")
        base = wid * b_per_w
        pltpu.sync_copy(ufT_hbm.at[pl.ds(0, 8), pl.ds(base, b_per_w)], uf_v)
        for jg in range(G):
            iv = uf_v[0, pl.ds(jg * _L, _L)]
            for b in range(_L):
                idx_s[0, jg * _L + b] = iv[b]

        def fetch(col, slot):
            col = pl.multiple_of(col, 128)
            for tr in range(4):
                pltpu.async_copy(
                    tT_hbm.at[tr, :, pl.ds(col, 128)],
                    blk_v.at[slot, tr],
                    sems.at[slot],
                )

        for b in range(_L):
            fetch((idx_s[0, b] // 128) * 128, b)

        def step(jg, carry):
            outc = [jnp.zeros((_L,), jnp.float32) for _ in range(D)]
            for b in range(_L):
                j = jg * _L + b
                pltpu.make_async_copy(
                    tT_hbm.at[0, :, pl.ds(0, 128)], blk_v.at[b, 0], sems.at[b]
                ).wait()
                pltpu.make_async_copy(
                    tT_hbm.at[0, :, pl.ds(0, 128)], blk_v.at[b, 0], sems.at[b]
                ).wait()
                pltpu.make_async_copy(
                    tT_hbm.at[0, :, pl.ds(0, 128)], blk_v.at[b, 0], sems.at[b]
                ).wait()
                pltpu.make_async_copy(
                    tT_hbm.at[0, :, pl.ds(0, 128)], blk_v.at[b, 0], sems.at[b]
                ).wait()
                lane = idx_s[0, j] % 128
                off = (lane // _L) * _L
                pidx = jnp.full((_L,), lane % _L, jnp.int32)
                bmask = lax.iota(jnp.int32, _L) == b
                for c in range(D):
                    v = blk_v[b, c // 8, c % 8, pl.ds(off, _L)]
                    val = v.at[pidx].get(mode="promise_in_bounds")
                    outc[c] = jnp.where(bmask, val, outc[c])

                @pl.when(jg + 1 < G)
                def _():
                    jn = jnp.minimum(j + _L, b_per_w - 1)
                    fetch((idx_s[0, jn] // 128) * 128, b)

            for c in range(D):
                w_v[c, pl.ds(jg * _L, _L)] = outc[c]
            return carry

        lax.fori_loop(0, G, step, 0, unroll=False)
        pltpu.sync_copy(w_v, outT_hbm.at[:, pl.ds(base, b_per_w)])

    return k


def kernel(user_fea, embedding_location):
    B, N_FEA = user_fea.shape
    V, D = embedding_location.shape
    k = _build(B, N_FEA, V, D)
    outT = k(user_fea.T, embedding_location.T.reshape(4, D // 4, V))
    return outT.T


# speedup vs baseline: 1.0111x; 1.0111x over previous
import functools

import jax
import jax.numpy as jnp
from jax import lax
from jax.experimental import pallas as pl
from jax.experimental.pallas import tpu as pltpu
from jax.experimental.pallas import tpu_sc as plsc

_L = 16


@functools.lru_cache(maxsize=None)
def _build(B, N_FEA, V, D):
    info = plsc.get_sparse_core_info()
    NC, NS = info.num_cores, info.num_subcores
    NW = NC * NS
    assert B % (8 * NW) == 0
    b_per_w = B // NW
    G = b_per_w // _L
    mesh = plsc.VectorSubcoreMesh(core_axis_name="c", subcore_axis_name="s")

    @functools.partial(
        pl.kernel,
        mesh=mesh,
        out_type=jax.ShapeDtypeStruct((D, B), jnp.float32),
        scratch_types=[
            pltpu.VMEM((8, b_per_w), jnp.int32),
            pltpu.SMEM((1, b_per_w), jnp.int32),
            pltpu.VMEM((_L, D, 128), jnp.float32),
            pltpu.VMEM((D, b_per_w), jnp.float32),
            pltpu.SemaphoreType.DMA((_L,)),
        ],
    )
    def k(ufT_hbm, tT_hbm, outT_hbm, uf_v, idx_s, blk_v, w_v, sems):
        wid = lax.axis_index("s") * NC + lax.axis_index("c")
        base = wid * b_per_w
        pltpu.sync_copy(ufT_hbm.at[pl.ds(0, 8), pl.ds(base, b_per_w)], uf_v)
        for jg in range(G):
            iv = uf_v[0, pl.ds(jg * _L, _L)]
            for b in range(_L):
                idx_s[0, jg * _L + b] = iv[b]

        def fetch(col, slot):
            col = pl.multiple_of(col, 128)
            return pltpu.async_copy(
                tT_hbm.at[:, pl.ds(col, 128)], blk_v.at[slot], sems.at[slot]
            )

        def col_of(j):
            return jnp.minimum((idx_s[0, j] // 128) * 128, V - 128)

        for b in range(_L):
            fetch(col_of(b), b)

        def step(jg, carry):
            outc = [jnp.zeros((_L,), jnp.float32) for _ in range(D)]
            for b in range(_L):
                j = jg * _L + b
                pltpu.make_async_copy(
                    tT_hbm.at[:, pl.ds(0, 128)], blk_v.at[b], sems.at[b]
                ).wait()
                lane = idx_s[0, j] - col_of(j)
                off = (lane // _L) * _L
                pidx = jnp.full((_L,), lane % _L, jnp.int32)
                bmask = lax.iota(jnp.int32, _L) == b
                for c in range(D):
                    v = blk_v[b, c, pl.ds(off, _L)]
                    val = v.at[pidx].get(mode="promise_in_bounds")
                    outc[c] = jnp.where(bmask, val, outc[c])

                @pl.when(jg + 1 < G)
                def _():
                    jn = jnp.minimum(j + _L, b_per_w - 1)
                    fetch(col_of(jn), b)

            for c in range(D):
                w_v[c, pl.ds(jg * _L, _L)] = outc[c]
            return carry

        lax.fori_loop(0, G, step, 0, unroll=False)
        pltpu.sync_copy(w_v, outT_hbm.at[:, pl.ds(base, b_per_w)])

    return k


def kernel(user_fea, embedding_location):
    B, N_FEA = user_fea.shape
    V, D = embedding_location.shape
    k = _build(B, N_FEA, V, D)
    outT = k(user_fea.T, embedding_location.T)
    return outT.T


# speedup vs baseline: 1.0130x; 1.0019x over previous
import functools

import jax
import jax.numpy as jnp
from jax import lax
from jax.experimental import pallas as pl
from jax.experimental.pallas import tpu as pltpu
from jax.experimental.pallas import tpu_sc as plsc

_L = 16


@functools.lru_cache(maxsize=None)
def _build(B, N_FEA, V, D):
    info = plsc.get_sparse_core_info()
    NC, NS = info.num_cores, info.num_subcores
    NW = NC * NS
    assert B % (8 * NW) == 0
    b_per_w = B // NW
    G = b_per_w // _L
    mesh = plsc.VectorSubcoreMesh(core_axis_name="c", subcore_axis_name="s")

    @functools.partial(
        pl.kernel,
        mesh=mesh,
        out_type=jax.ShapeDtypeStruct((D, B), jnp.float32),
        scratch_types=[
            pltpu.VMEM((8, b_per_w), jnp.int32),
            pltpu.SMEM((1, b_per_w), jnp.int32),
            pltpu.VMEM((_L, D, 128), jnp.float32),
            pltpu.VMEM((D, b_per_w), jnp.float32),
            pltpu.SemaphoreType.DMA((_L,)),
        ],
    )
    def k(ufT_hbm, tT_hbm, outT_hbm, uf_v, idx_s, blk_v, w_v, sems):
        wid = lax.axis_index("s") * NC + lax.axis_index("c")
        base = wid * b_per_w
        pltpu.sync_copy(ufT_hbm.at[pl.ds(0, 8), pl.ds(base, b_per_w)], uf_v)
        for jg in range(G):
            iv = uf_v[0, pl.ds(jg * _L, _L)]
            for b in range(_L):
                idx_s[0, jg * _L + b] = iv[b]

        def fetch(col, slot):
            col = pl.multiple_of(col, 128)
            return pltpu.async_copy(
                tT_hbm.at[:, pl.ds(col, 128)], blk_v.at[slot], sems.at[slot]
            )

        for b in range(_L):
            fetch((idx_s[0, b] // 128) * 128, b)

        def step(jg, carry):
            outc = [jnp.zeros((_L,), jnp.float32) for _ in range(D)]
            for b in range(_L):
                j = jg * _L + b
                pltpu.make_async_copy(
                    tT_hbm.at[:, pl.ds(0, 128)], blk_v.at[b], sems.at[b]
                ).wait()
                lane = idx_s[0, j] % 128
                off = (lane // _L) * _L
                pidx = jnp.full((_L,), lane % _L, jnp.int32)
                bmask = lax.iota(jnp.int32, _L) == b
                for c in range(D):
                    v = blk_v[b, c, pl.ds(off, _L)]
                    val = v.at[pidx].get(mode="promise_in_bounds")
                    outc[c] = jnp.where(bmask, val, outc[c])

                @pl.when(jg + 1 < G)
                def _():
                    jn = jnp.minimum(j + _L, b_per_w - 1)
                    fetch((idx_s[0, jn] // 128) * 128, b)

            for c in range(D):
                w_v[c, pl.ds(jg * _L, _L)] = outc[c]
            return carry

        lax.fori_loop(0, G, step, 0, unroll=False)
        pltpu.sync_copy(w_v, outT_hbm.at[:, pl.ds(base, b_per_w)])

    return k


def kernel(user_fea, embedding_location):
    B, N_FEA = user_fea.shape
    V, D = embedding_location.shape
    k = _build(B, N_FEA, V, D)
    outT = k(user_fea.T, embedding_location.T)
    return outT.T
